# Initial kernel scaffold; baseline (speedup 1.0000x reference)
import jax
import jax.numpy as jnp
from jax.experimental import pallas as pl


def kernel(x, edge_index, W1, b1, W2, b2):
    raise NotImplementedError("write your pallas kernel here")


# speedup vs baseline: 14.7058x; 14.7058x over previous
import functools

import jax
import jax.numpy as jnp
from jax import lax
from jax.experimental import pallas as pl
from jax.experimental.pallas import tpu as pltpu
from jax.experimental.pallas import tpu_sc as plsc

NC = 2
NS = 16
NW = NC * NS
CHUNK = 128
BLK = 1000
LANES = 16
ZROWS = 64


def _sc_deg_body(chunks, rows_per_tile, dst_hbm, out_hbm, idx_v, ones_v,
                 zrow_v, shared_deg):
  cid = lax.axis_index("c")
  sid = lax.axis_index("s")
  wid = cid * NS + sid

  def initc(i, _):
    ones_v[pl.ds(i * LANES, LANES)] = jnp.ones((LANES,), jnp.float32)
    return _

  lax.fori_loop(0, CHUNK // LANES, initc, None)

  def initz(i, _):
    zrow_v[pl.ds(i * LANES, LANES)] = jnp.zeros((LANES,), jnp.float32)
    return _

  lax.fori_loop(0, rows_per_tile // LANES, initz, None)
  pltpu.sync_copy(zrow_v, shared_deg.at[pl.ds(sid * rows_per_tile,
                                              rows_per_tile)])
  plsc.subcore_barrier()

  base = wid * (chunks * CHUNK)

  def step(k, _):
    pltpu.sync_copy(dst_hbm.at[pl.ds(base + k * CHUNK, CHUNK)], idx_v)
    pltpu.sync_copy(ones_v, shared_deg.at[idx_v], add=True)
    return _

  lax.fori_loop(0, chunks, step, None)
  plsc.subcore_barrier()
  n_pad = rows_per_tile * NS
  pltpu.sync_copy(shared_deg.at[pl.ds(sid * rows_per_tile, rows_per_tile)],
                  zrow_v)
  pltpu.sync_copy(
      zrow_v,
      out_hbm.at[pl.ds(cid * n_pad + sid * rows_per_tile, rows_per_tile)])


def _sc_agg_body(chunks, rows_per_tile, d_hid, src_hbm, dst_hbm, hs_hbm,
                 out_hbm, sidx_v, didx_v, rows_v, zbuf_v, gsem, shared_acc):
  cid = lax.axis_index("c")
  sid = lax.axis_index("s")
  wid = cid * NS + sid

  per_row = d_hid // LANES

  def initz(i, _):
    zbuf_v[i // per_row, pl.ds((i % per_row) * LANES, LANES)] = (
        jnp.zeros((LANES,), jnp.float32))
    return _

  lax.fori_loop(0, ZROWS * per_row, initz, None)

  def zcopy(j, _):
    pltpu.sync_copy(
        zbuf_v, shared_acc.at[pl.ds(sid * rows_per_tile + j * ZROWS, ZROWS), :])
    return _

  lax.fori_loop(0, rows_per_tile // ZROWS, zcopy, None)
  plsc.subcore_barrier()

  base = wid * (chunks * CHUNK)

  def step(k, _):
    off = base + k * CHUNK
    pltpu.sync_copy(src_hbm.at[pl.ds(off, CHUNK)], sidx_v)
    pltpu.sync_copy(dst_hbm.at[pl.ds(off, CHUNK)], didx_v)
    pltpu.async_copy(hs_hbm.at[sidx_v], rows_v, gsem).wait()
    pltpu.sync_copy(rows_v, shared_acc.at[didx_v], add=True)
    return _

  lax.fori_loop(0, chunks, step, None)
  plsc.subcore_barrier()

  def ocopy(j, _):
    row0 = sid * rows_per_tile + j * ZROWS
    pltpu.sync_copy(shared_acc.at[pl.ds(row0, ZROWS), :], zbuf_v)
    pltpu.sync_copy(zbuf_v, out_hbm.at[cid, pl.ds(row0, ZROWS), :])
    return _

  lax.fori_loop(0, rows_per_tile // ZROWS, ocopy, None)


def _tc_hs_body(x_ref, w1_ref, degt_ref, hs_ref):
  deg = degt_ref[:, 0] + degt_ref[:, 1] + 1.0
  dinv = 1.0 / jnp.sqrt(deg)
  h = jnp.dot(x_ref[...], w1_ref[...], preferred_element_type=jnp.float32)
  hs_ref[...] = h * dinv[:, None]


def _tc_out_body(a0_ref, a1_ref, hs_ref, degt_ref, b1_ref, w2_ref, b2_ref,
                 out_ref):
  deg = degt_ref[:, 0] + degt_ref[:, 1] + 1.0
  dinv = 1.0 / jnp.sqrt(deg)
  hs = hs_ref[...]
  t = (a0_ref[0] + a1_ref[0] + hs) * dinv[:, None] + b1_ref[...]
  t = jnp.maximum(t, 0.0)
  out_ref[...] = jnp.dot(t, w2_ref[...],
                         preferred_element_type=jnp.float32) + b2_ref[...]


def kernel(x, edge_index, W1, b1, W2, b2):
  n = x.shape[0]
  e = edge_index.shape[1]
  d_in = x.shape[1]
  d_hid = W1.shape[1]
  d_out = W2.shape[1]

  rows_per_tile = -(-(n + 1) // (NS * ZROWS)) * ZROWS
  n_pad = rows_per_tile * NS

  src = edge_index[0].astype(jnp.int32)
  dst = edge_index[1].astype(jnp.int32)
  chunks = -(-e // (NW * CHUNK))
  e_pad = NW * chunks * CHUNK
  pad = e_pad - e
  src_p = jnp.concatenate([src, jnp.zeros((pad,), jnp.int32)])
  dst_p = jnp.concatenate([dst, jnp.full((pad,), n, jnp.int32)])

  mesh = plsc.VectorSubcoreMesh(core_axis_name="c", subcore_axis_name="s")

  sc_deg = pl.kernel(
      functools.partial(_sc_deg_body, chunks, rows_per_tile),
      out_type=jax.ShapeDtypeStruct((NC * n_pad,), jnp.float32),
      mesh=mesh,
      scratch_types=[
          pltpu.VMEM((CHUNK,), jnp.int32),
          pltpu.VMEM((CHUNK,), jnp.float32),
          pltpu.VMEM((rows_per_tile,), jnp.float32),
          pltpu.VMEM_SHARED((n_pad,), jnp.float32),
      ],
  )
  degp = sc_deg(dst_p).reshape(NC, n_pad)
  degt = degp.T

  grid = n // BLK
  tc_hs = pl.pallas_call(
      _tc_hs_body,
      grid=(grid,),
      in_specs=[
          pl.BlockSpec((BLK, d_in), lambda i: (i, 0)),
          pl.BlockSpec((d_in, d_hid), lambda i: (0, 0)),
          pl.BlockSpec((BLK, NC), lambda i: (i, 0)),
      ],
      out_specs=pl.BlockSpec((BLK, d_hid), lambda i: (i, 0)),
      out_shape=jax.ShapeDtypeStruct((n, d_hid), jnp.float32),
  )
  hs = tc_hs(x, W1, degt)

  sc_agg = pl.kernel(
      functools.partial(_sc_agg_body, chunks, rows_per_tile, d_hid),
      out_type=jax.ShapeDtypeStruct((NC, n_pad, d_hid), jnp.float32),
      mesh=mesh,
      scratch_types=[
          pltpu.VMEM((CHUNK,), jnp.int32),
          pltpu.VMEM((CHUNK,), jnp.int32),
          pltpu.VMEM((CHUNK, d_hid), jnp.float32),
          pltpu.VMEM((ZROWS, d_hid), jnp.float32),
          pltpu.SemaphoreType.DMA,
          pltpu.VMEM_SHARED((n_pad, d_hid), jnp.float32),
      ],
  )
  aggp = sc_agg(src_p, dst_p, hs)

  tc_out = pl.pallas_call(
      _tc_out_body,
      grid=(grid,),
      in_specs=[
          pl.BlockSpec((1, BLK, d_hid), lambda i: (0, i, 0)),
          pl.BlockSpec((1, BLK, d_hid), lambda i: (1, i, 0)),
          pl.BlockSpec((BLK, d_hid), lambda i: (i, 0)),
          pl.BlockSpec((BLK, NC), lambda i: (i, 0)),
          pl.BlockSpec((d_hid,), lambda i: (0,)),
          pl.BlockSpec((d_hid, d_out), lambda i: (0, 0)),
          pl.BlockSpec((d_out,), lambda i: (0,)),
      ],
      out_specs=pl.BlockSpec((BLK, d_out), lambda i: (i, 0)),
      out_shape=jax.ShapeDtypeStruct((n, d_out), jnp.float32),
  )
  return tc_out(aggp, aggp, hs, degt, b1, W2, b2)
